# baseline (speedup 1.0000x reference)
import functools

import jax
import jax.numpy as jnp
from jax import lax
from jax.experimental import pallas as pl
from jax.experimental.pallas import tpu as pltpu
from jax.experimental.pallas import tpu_sc as plsc

D = 64
BB = 128

_info = plsc.get_sparse_core_info()
_NC = _info.num_cores
_NS = _info.num_subcores
_NW = _NC * _NS


@functools.lru_cache(maxsize=None)
def _build(b, s, v):
    nb_w = b // _NW
    n_blocks = s * (nb_w // BB)
    assert n_blocks % 2 == 0
    nbb = nb_w // BB
    mesh = plsc.VectorSubcoreMesh(core_axis_name="c", subcore_axis_name="s")

    @functools.partial(
        pl.kernel,
        mesh=mesh,
        compiler_params=pltpu.CompilerParams(
            use_tc_tiling_on_sc=True, needs_layout_passes=False),
        out_type=jax.ShapeDtypeStruct((s, D, b), jnp.float32),
        scratch_types=[
            pltpu.VMEM((s, nb_w), jnp.int32),
            pltpu.VMEM((BB, 128), jnp.float32),
            pltpu.VMEM((BB, 128), jnp.float32),
            pltpu.VMEM((D, BB), jnp.float32),
            pltpu.VMEM((D, BB), jnp.float32),
            pltpu.SemaphoreType.DMA,
            pltpu.SemaphoreType.DMA,
            pltpu.SemaphoreType.DMA,
            pltpu.SemaphoreType.DMA,
        ],
    )
    def emb(ids_hbm, table_hbm, out_hbm, idx_v, gbuf_a, gbuf_b,
            tbuf_a, tbuf_b, gsem_a, gsem_b, wsem_a, wsem_b):
        wid = lax.axis_index("s") * _NC + lax.axis_index("c")
        col0 = wid * nb_w
        pltpu.sync_copy(ids_hbm.at[:, pl.ds(col0, nb_w)], idx_v)

        def split(j):
            return j // nbb, (j % nbb) * BB

        def fire_gather(j, gbuf, sem):
            sq, bo = split(j)
            pltpu.async_copy(
                table_hbm.at[idx_v.at[sq, pl.ds(bo, BB)]], gbuf, sem)

        def wait_gather(gbuf, sem):
            pltpu.make_async_copy(
                table_hbm.at[pl.ds(0, BB)], gbuf, sem).wait()

        lanes = lax.iota(jnp.int32, 16)
        diag = [jnp.bitwise_and(lanes + d, 15) for d in range(16)]

        def transpose(gbuf, tbuf):
            @pl.loop(0, BB, step=16)
            def _(r0):
                rows = lanes + r0
                for f0 in range(0, D, 16):
                    for d in range(16):
                        cols = diag[d] + f0
                        vals = plsc.load_gather(gbuf, [rows, cols])
                        plsc.store_scatter(tbuf, [cols, rows], vals)

        def fire_write(j, tbuf, sem):
            sq, bo = split(j)
            pltpu.async_copy(
                tbuf, out_hbm.at[sq, :, pl.ds(col0 + bo, BB)], sem)

        def wait_write(tbuf, sem):
            pltpu.make_async_copy(
                tbuf, out_hbm.at[0, pl.ds(0, D), pl.ds(0, BB)], sem).wait()

        fire_gather(0, gbuf_a, gsem_a)

        @pl.loop(0, n_blocks, step=2)
        def body(ja):
            jb = ja + 1
            wait_gather(gbuf_a, gsem_a)
            fire_gather(jb, gbuf_b, gsem_b)

            @pl.when(ja > 0)
            def _():
                wait_write(tbuf_a, wsem_a)

            transpose(gbuf_a, tbuf_a)
            fire_write(ja, tbuf_a, wsem_a)
            wait_gather(gbuf_b, gsem_b)

            @pl.when(jb + 1 < n_blocks)
            def _():
                fire_gather(ja + 2, gbuf_a, gsem_a)

            @pl.when(ja > 0)
            def _():
                wait_write(tbuf_b, wsem_b)

            transpose(gbuf_b, tbuf_b)
            fire_write(jb, tbuf_b, wsem_b)

        wait_write(tbuf_a, wsem_a)
        wait_write(tbuf_b, wsem_b)

    return emb


def kernel(input_ids, table):
    b, s = input_ids.shape
    v = table.shape[0]
    table2 = jnp.pad(table, ((0, 0), (0, 128 - D)))
    out = _build(b, s, v)(input_ids.T, table2)
    return out.transpose(2, 0, 1)


# speedup vs baseline: 1.0058x; 1.0058x over previous
import functools

import jax
import jax.numpy as jnp
from jax import lax
from jax.experimental import pallas as pl
from jax.experimental.pallas import tpu as pltpu
from jax.experimental.pallas import tpu_sc as plsc

D = 64
BB = 128

_info = plsc.get_sparse_core_info()
_NC = _info.num_cores
_NS = _info.num_subcores
_NW = _NC * _NS

_PARAMS = pltpu.CompilerParams(
    use_tc_tiling_on_sc=True, needs_layout_passes=False,
    disable_bounds_checks=True)


def _diag_vectors():
    lanes = lax.iota(jnp.int32, 16)
    return lanes, [jnp.bitwise_and(lanes + d, 15) for d in range(16)]


@functools.lru_cache(maxsize=None)
def _build_prep(v):
    n_blocks = (v + BB - 1) // BB
    per_w = (n_blocks + _NW - 1) // _NW
    last_v0 = (n_blocks - 1) * BB
    v_pad = n_blocks * BB
    mesh = plsc.VectorSubcoreMesh(core_axis_name="c", subcore_axis_name="s")

    @functools.partial(
        pl.kernel,
        mesh=mesh,
        compiler_params=_PARAMS,
        out_type=jax.ShapeDtypeStruct((v_pad, BB), jnp.float32),
        scratch_types=[
            pltpu.VMEM((D, BB), jnp.float32),
            pltpu.VMEM((D, BB), jnp.float32),
            pltpu.VMEM((BB, BB), jnp.float32),
            pltpu.VMEM((BB, BB), jnp.float32),
            pltpu.SemaphoreType.DMA,
            pltpu.SemaphoreType.DMA,
            pltpu.SemaphoreType.DMA,
            pltpu.SemaphoreType.DMA,
        ],
    )
    def prep(tt_hbm, tp_hbm, rbuf_a, rbuf_b, wbuf_a, wbuf_b,
             rsem_a, rsem_b, wsem_a, wsem_b):
        wid = lax.axis_index("s") * _NC + lax.axis_index("c")
        lanes, diag = _diag_vectors()

        def v0_of(j):
            v0 = jnp.minimum((wid + j * _NW) * BB, last_v0)
            return pl.multiple_of(v0, BB)

        def fire_read(j, rbuf, sem):
            pltpu.async_copy(tt_hbm.at[:, pl.ds(v0_of(j), BB)], rbuf, sem)

        def wait_read(rbuf, sem):
            pltpu.make_async_copy(
                tt_hbm.at[:, pl.ds(0, BB)], rbuf, sem).wait()

        def transpose(rbuf, wbuf):
            @pl.loop(0, BB, step=16)
            def _(vv0):
                for f0 in range(0, D, 16):
                    rows = lanes + f0
                    for d in range(16):
                        cols = diag[d] + vv0
                        vals = plsc.load_gather(rbuf, [rows, cols])
                        plsc.store_scatter(wbuf, [cols, rows], vals)

        def fire_write(j, wbuf, sem):
            pltpu.async_copy(wbuf, tp_hbm.at[pl.ds(v0_of(j), BB)], sem)

        def wait_write(wbuf, sem):
            pltpu.make_async_copy(
                wbuf, tp_hbm.at[pl.ds(0, BB)], sem).wait()

        fire_read(0, rbuf_a, rsem_a)

        @pl.loop(0, per_w, step=2)
        def body(ja):
            jb = ja + 1
            wait_read(rbuf_a, rsem_a)
            fire_read(jb, rbuf_b, rsem_b)

            @pl.when(ja > 0)
            def _():
                wait_write(wbuf_a, wsem_a)

            transpose(rbuf_a, wbuf_a)
            fire_write(ja, wbuf_a, wsem_a)
            wait_read(rbuf_b, rsem_b)

            @pl.when(jb + 1 < per_w)
            def _():
                fire_read(ja + 2, rbuf_a, rsem_a)

            @pl.when(ja > 0)
            def _():
                wait_write(wbuf_b, wsem_b)

            transpose(rbuf_b, wbuf_b)
            fire_write(jb, wbuf_b, wsem_b)

        wait_write(wbuf_a, wsem_a)
        wait_write(wbuf_b, wsem_b)

    return prep


@functools.lru_cache(maxsize=None)
def _build_lookup(b, s, v):
    nb_w = b // _NW
    n_blocks = s * (nb_w // BB)
    assert n_blocks % 2 == 0
    nbb = nb_w // BB
    mesh = plsc.VectorSubcoreMesh(core_axis_name="c", subcore_axis_name="s")

    @functools.partial(
        pl.kernel,
        mesh=mesh,
        compiler_params=_PARAMS,
        out_type=jax.ShapeDtypeStruct((s, D, b), jnp.float32),
        scratch_types=[
            pltpu.VMEM((s, nb_w), jnp.int32),
            pltpu.VMEM((BB, 128), jnp.float32),
            pltpu.VMEM((BB, 128), jnp.float32),
            pltpu.VMEM((D, BB), jnp.float32),
            pltpu.VMEM((D, BB), jnp.float32),
            pltpu.SemaphoreType.DMA,
            pltpu.SemaphoreType.DMA,
            pltpu.SemaphoreType.DMA,
            pltpu.SemaphoreType.DMA,
        ],
    )
    def emb(ids_hbm, table_hbm, out_hbm, idx_v, gbuf_a, gbuf_b,
            tbuf_a, tbuf_b, gsem_a, gsem_b, wsem_a, wsem_b):
        wid = lax.axis_index("s") * _NC + lax.axis_index("c")
        col0 = wid * nb_w
        lanes, diag = _diag_vectors()
        pltpu.sync_copy(ids_hbm.at[:, pl.ds(col0, nb_w)], idx_v)

        def split(j):
            return j // nbb, (j % nbb) * BB

        def fire_gather(j, gbuf, sem):
            sq, bo = split(j)
            pltpu.async_copy(
                table_hbm.at[idx_v.at[sq, pl.ds(bo, BB)]], gbuf, sem)

        def wait_gather(gbuf, sem):
            pltpu.make_async_copy(
                table_hbm.at[pl.ds(0, BB)], gbuf, sem).wait()

        def transpose(gbuf, tbuf):
            @pl.loop(0, BB, step=16)
            def _(r0):
                rows = lanes + r0
                for f0 in range(0, D, 16):
                    for d in range(16):
                        cols = diag[d] + f0
                        vals = plsc.load_gather(gbuf, [rows, cols])
                        plsc.store_scatter(tbuf, [cols, rows], vals)

        def fire_write(j, tbuf, sem):
            sq, bo = split(j)
            pltpu.async_copy(
                tbuf, out_hbm.at[sq, :, pl.ds(col0 + bo, BB)], sem)

        def wait_write(tbuf, sem):
            pltpu.make_async_copy(
                tbuf, out_hbm.at[0, pl.ds(0, D), pl.ds(0, BB)], sem).wait()

        fire_gather(0, gbuf_a, gsem_a)

        @pl.loop(0, n_blocks, step=2)
        def body(ja):
            jb = ja + 1
            wait_gather(gbuf_a, gsem_a)
            fire_gather(jb, gbuf_b, gsem_b)

            @pl.when(ja > 0)
            def _():
                wait_write(tbuf_a, wsem_a)

            transpose(gbuf_a, tbuf_a)
            fire_write(ja, tbuf_a, wsem_a)
            wait_gather(gbuf_b, gsem_b)

            @pl.when(jb + 1 < n_blocks)
            def _():
                fire_gather(ja + 2, gbuf_a, gsem_a)

            @pl.when(ja > 0)
            def _():
                wait_write(tbuf_b, wsem_b)

            transpose(gbuf_b, tbuf_b)
            fire_write(jb, tbuf_b, wsem_b)

        wait_write(tbuf_a, wsem_a)
        wait_write(tbuf_b, wsem_b)

    return emb


def kernel(input_ids, table):
    b, s = input_ids.shape
    v = table.shape[0]
    table2 = _build_prep(v)(table.T)
    out = _build_lookup(b, s, v)(input_ids.T, table2)
    return out.transpose(2, 0, 1)


# speedup vs baseline: 1.0142x; 1.0084x over previous
import functools

import jax
import jax.numpy as jnp
from jax import lax
from jax.experimental import pallas as pl
from jax.experimental.pallas import tpu as pltpu
from jax.experimental.pallas import tpu_sc as plsc

D = 64
BB = 128

_info = plsc.get_sparse_core_info()
_NC = _info.num_cores
_NS = _info.num_subcores
_NW = _NC * _NS

_PARAMS = pltpu.CompilerParams(
    use_tc_tiling_on_sc=True, needs_layout_passes=False,
    disable_bounds_checks=True)


def _diag_vectors():
    lanes = lax.iota(jnp.int32, 16)
    return lanes, [jnp.bitwise_and(lanes + d, 15) for d in range(16)]


@functools.lru_cache(maxsize=None)
def _build_prep(v):
    vb = 2 * BB
    v_phys = ((v + BB - 1) // BB) * BB
    v_pad = v_phys
    full = v_phys // vb
    n_blocks = full + (1 if v_phys % vb else 0)
    last_v0 = v_phys - vb
    per_w = (n_blocks + _NW - 1) // _NW
    mesh = plsc.VectorSubcoreMesh(core_axis_name="c", subcore_axis_name="s")

    @functools.partial(
        pl.kernel,
        mesh=mesh,
        compiler_params=_PARAMS,
        out_type=jax.ShapeDtypeStruct((v_pad, BB), jnp.float32),
        scratch_types=[
            pltpu.VMEM((D, vb), jnp.float32),
            pltpu.VMEM((D, vb), jnp.float32),
            pltpu.VMEM((vb, BB), jnp.float32),
            pltpu.VMEM((vb, BB), jnp.float32),
            pltpu.SemaphoreType.DMA,
            pltpu.SemaphoreType.DMA,
            pltpu.SemaphoreType.DMA,
            pltpu.SemaphoreType.DMA,
        ],
    )
    def prep(tt_hbm, tp_hbm, rbuf_a, rbuf_b, wbuf_a, wbuf_b,
             rsem_a, rsem_b, wsem_a, wsem_b):
        wid = lax.axis_index("s") * _NC + lax.axis_index("c")
        lanes, diag = _diag_vectors()

        def v0_of(j):
            v0 = jnp.minimum((wid + j * _NW) * vb, last_v0)
            return pl.multiple_of(v0, BB)

        def fire_read(j, rbuf, sem):
            pltpu.async_copy(tt_hbm.at[:, pl.ds(v0_of(j), vb)], rbuf, sem)

        def wait_read(rbuf, sem):
            pltpu.make_async_copy(
                tt_hbm.at[:, pl.ds(0, vb)], rbuf, sem).wait()

        def transpose(rbuf, wbuf):
            @pl.loop(0, vb, step=16)
            def _(vv0):
                for f0 in range(0, D, 16):
                    rows = lanes + f0
                    for d in range(16):
                        cols = diag[d] + vv0
                        vals = plsc.load_gather(rbuf, [rows, cols])
                        plsc.store_scatter(wbuf, [cols, rows], vals)

        def fire_write(j, wbuf, sem):
            pltpu.async_copy(wbuf, tp_hbm.at[pl.ds(v0_of(j), vb)], sem)

        def wait_write(wbuf, sem):
            pltpu.make_async_copy(
                wbuf, tp_hbm.at[pl.ds(0, vb)], sem).wait()

        fire_read(0, rbuf_a, rsem_a)

        @pl.loop(0, per_w, step=2)
        def body(ja):
            jb = ja + 1
            wait_read(rbuf_a, rsem_a)
            fire_read(jb, rbuf_b, rsem_b)

            @pl.when(ja > 0)
            def _():
                wait_write(wbuf_a, wsem_a)

            transpose(rbuf_a, wbuf_a)
            fire_write(ja, wbuf_a, wsem_a)
            wait_read(rbuf_b, rsem_b)

            @pl.when(jb + 1 < per_w)
            def _():
                fire_read(ja + 2, rbuf_a, rsem_a)

            @pl.when(ja > 0)
            def _():
                wait_write(wbuf_b, wsem_b)

            transpose(rbuf_b, wbuf_b)
            fire_write(jb, wbuf_b, wsem_b)

        wait_write(wbuf_a, wsem_a)
        wait_write(wbuf_b, wsem_b)

    return prep


@functools.lru_cache(maxsize=None)
def _build_lookup(b, s, v):
    nb_w = b // _NW
    n_blocks = s * (nb_w // BB)
    assert n_blocks % 2 == 0
    nbb = nb_w // BB
    mesh = plsc.VectorSubcoreMesh(core_axis_name="c", subcore_axis_name="s")

    @functools.partial(
        pl.kernel,
        mesh=mesh,
        compiler_params=_PARAMS,
        out_type=jax.ShapeDtypeStruct((s, D, b), jnp.float32),
        scratch_types=[
            pltpu.VMEM((s, nb_w), jnp.int32),
            pltpu.VMEM((BB, 128), jnp.float32),
            pltpu.VMEM((BB, 128), jnp.float32),
            pltpu.VMEM((D, BB), jnp.float32),
            pltpu.VMEM((D, BB), jnp.float32),
            pltpu.SemaphoreType.DMA,
            pltpu.SemaphoreType.DMA,
            pltpu.SemaphoreType.DMA,
            pltpu.SemaphoreType.DMA,
        ],
    )
    def emb(ids_hbm, table_hbm, out_hbm, idx_v, gbuf_a, gbuf_b,
            tbuf_a, tbuf_b, gsem_a, gsem_b, wsem_a, wsem_b):
        wid = lax.axis_index("s") * _NC + lax.axis_index("c")
        col0 = wid * nb_w
        lanes, diag = _diag_vectors()
        pltpu.sync_copy(ids_hbm.at[:, pl.ds(col0, nb_w)], idx_v)

        def split(j):
            return j // nbb, (j % nbb) * BB

        def fire_gather(j, gbuf, sem):
            sq, bo = split(j)
            pltpu.async_copy(
                table_hbm.at[idx_v.at[sq, pl.ds(bo, BB)]], gbuf, sem)

        def wait_gather(gbuf, sem):
            pltpu.make_async_copy(
                table_hbm.at[pl.ds(0, BB)], gbuf, sem).wait()

        def transpose(gbuf, tbuf):
            @pl.loop(0, BB, step=16)
            def _(r0):
                rows = lanes + r0
                for f0 in range(0, D, 16):
                    for d in range(16):
                        cols = diag[d] + f0
                        vals = plsc.load_gather(gbuf, [rows, cols])
                        plsc.store_scatter(tbuf, [cols, rows], vals)

        def fire_write(j, tbuf, sem):
            sq, bo = split(j)
            pltpu.async_copy(
                tbuf, out_hbm.at[sq, :, pl.ds(col0 + bo, BB)], sem)

        def wait_write(tbuf, sem):
            pltpu.make_async_copy(
                tbuf, out_hbm.at[0, pl.ds(0, D), pl.ds(0, BB)], sem).wait()

        fire_gather(0, gbuf_a, gsem_a)

        @pl.loop(0, n_blocks, step=2)
        def body(ja):
            jb = ja + 1
            wait_gather(gbuf_a, gsem_a)
            fire_gather(jb, gbuf_b, gsem_b)

            @pl.when(ja > 0)
            def _():
                wait_write(tbuf_a, wsem_a)

            transpose(gbuf_a, tbuf_a)
            fire_write(ja, tbuf_a, wsem_a)
            wait_gather(gbuf_b, gsem_b)

            @pl.when(jb + 1 < n_blocks)
            def _():
                fire_gather(ja + 2, gbuf_a, gsem_a)

            @pl.when(ja > 0)
            def _():
                wait_write(tbuf_b, wsem_b)

            transpose(gbuf_b, tbuf_b)
            fire_write(jb, tbuf_b, wsem_b)

        wait_write(tbuf_a, wsem_a)
        wait_write(tbuf_b, wsem_b)

    return emb


def kernel(input_ids, table):
    b, s = input_ids.shape
    v = table.shape[0]
    table2 = _build_prep(v)(table.T)
    out = _build_lookup(b, s, v)(input_ids.T, table2)
    return out.transpose(2, 0, 1)


# speedup vs baseline: 1.1455x; 1.1295x over previous
import functools

import jax
import jax.numpy as jnp
from jax import lax
from jax.experimental import pallas as pl
from jax.experimental.pallas import tpu as pltpu
from jax.experimental.pallas import tpu_sc as plsc

D = 64
BB = 128

_info = plsc.get_sparse_core_info()
_NC = _info.num_cores
_NS = _info.num_subcores
_NW = _NC * _NS

_PARAMS = pltpu.CompilerParams(
    use_tc_tiling_on_sc=True, needs_layout_passes=False,
    disable_bounds_checks=True)


def _diag_vectors():
    lanes = lax.iota(jnp.int32, 16)
    return lanes, [jnp.bitwise_and(lanes + d, 15) for d in range(16)]


@functools.lru_cache(maxsize=None)
def _build_prep(v):
    vb = 2 * BB
    v_phys = ((v + BB - 1) // BB) * BB
    v_pad = v_phys
    full = v_phys // vb
    n_blocks = full + (1 if v_phys % vb else 0)
    last_v0 = v_phys - vb
    per_w = (n_blocks + _NW - 1) // _NW
    mesh = plsc.VectorSubcoreMesh(core_axis_name="c", subcore_axis_name="s")

    @functools.partial(
        pl.kernel,
        mesh=mesh,
        compiler_params=_PARAMS,
        out_type=jax.ShapeDtypeStruct((v_pad, BB), jnp.float32),
        scratch_types=[
            pltpu.VMEM((D, vb), jnp.float32),
            pltpu.VMEM((D, vb), jnp.float32),
            pltpu.VMEM((vb, BB), jnp.float32),
            pltpu.VMEM((vb, BB), jnp.float32),
            pltpu.SemaphoreType.DMA,
            pltpu.SemaphoreType.DMA,
            pltpu.SemaphoreType.DMA,
            pltpu.SemaphoreType.DMA,
        ],
    )
    def prep(tt_hbm, tp_hbm, rbuf_a, rbuf_b, wbuf_a, wbuf_b,
             rsem_a, rsem_b, wsem_a, wsem_b):
        wid = lax.axis_index("s") * _NC + lax.axis_index("c")
        lanes, diag = _diag_vectors()

        def v0_of(j):
            v0 = jnp.minimum((wid + j * _NW) * vb, last_v0)
            return pl.multiple_of(v0, BB)

        def fire_read(j, rbuf, sem):
            pltpu.async_copy(tt_hbm.at[:, pl.ds(v0_of(j), vb)], rbuf, sem)

        def wait_read(rbuf, sem):
            pltpu.make_async_copy(
                tt_hbm.at[:, pl.ds(0, vb)], rbuf, sem).wait()

        def transpose(rbuf, wbuf):
            @pl.loop(0, vb, step=16, unroll=2)
            def _(vv0):
                for f0 in range(0, D, 16):
                    rows = lanes + f0
                    for d in range(16):
                        cols = diag[d] + vv0
                        vals = plsc.load_gather(rbuf, [rows, cols])
                        plsc.store_scatter(wbuf, [cols, rows], vals)

        def fire_write(j, wbuf, sem):
            pltpu.async_copy(wbuf, tp_hbm.at[pl.ds(v0_of(j), vb)], sem)

        def wait_write(wbuf, sem):
            pltpu.make_async_copy(
                wbuf, tp_hbm.at[pl.ds(0, vb)], sem).wait()

        fire_read(0, rbuf_a, rsem_a)

        @pl.loop(0, per_w, step=2)
        def body(ja):
            jb = ja + 1
            wait_read(rbuf_a, rsem_a)
            fire_read(jb, rbuf_b, rsem_b)

            @pl.when(ja > 0)
            def _():
                wait_write(wbuf_a, wsem_a)

            transpose(rbuf_a, wbuf_a)
            fire_write(ja, wbuf_a, wsem_a)
            wait_read(rbuf_b, rsem_b)

            @pl.when(jb + 1 < per_w)
            def _():
                fire_read(ja + 2, rbuf_a, rsem_a)

            @pl.when(ja > 0)
            def _():
                wait_write(wbuf_b, wsem_b)

            transpose(rbuf_b, wbuf_b)
            fire_write(jb, wbuf_b, wsem_b)

        wait_write(wbuf_a, wsem_a)
        wait_write(wbuf_b, wsem_b)

    return prep


@functools.lru_cache(maxsize=None)
def _build_lookup(b, s, v):
    nb_w = b // _NW
    n_blocks = s * (nb_w // BB)
    assert n_blocks % 2 == 0
    nbb = nb_w // BB
    mesh = plsc.VectorSubcoreMesh(core_axis_name="c", subcore_axis_name="s")

    @functools.partial(
        pl.kernel,
        mesh=mesh,
        compiler_params=_PARAMS,
        out_type=jax.ShapeDtypeStruct((s, D, b), jnp.float32),
        scratch_types=[
            pltpu.VMEM((s, nb_w), jnp.int32),
            pltpu.VMEM((BB, 128), jnp.float32),
            pltpu.VMEM((BB, 128), jnp.float32),
            pltpu.VMEM((D, BB), jnp.float32),
            pltpu.VMEM((D, BB), jnp.float32),
            pltpu.SemaphoreType.DMA,
            pltpu.SemaphoreType.DMA,
            pltpu.SemaphoreType.DMA,
            pltpu.SemaphoreType.DMA,
        ],
    )
    def emb(ids_hbm, table_hbm, out_hbm, idx_v, gbuf_a, gbuf_b,
            tbuf_a, tbuf_b, gsem_a, gsem_b, wsem_a, wsem_b):
        wid = lax.axis_index("s") * _NC + lax.axis_index("c")
        col0 = wid * nb_w
        lanes, diag = _diag_vectors()
        pltpu.sync_copy(ids_hbm.at[:, pl.ds(col0, nb_w)], idx_v)

        def split(j):
            return j // nbb, (j % nbb) * BB

        def fire_gather(j, gbuf, sem):
            sq, bo = split(j)
            pltpu.async_copy(
                table_hbm.at[idx_v.at[sq, pl.ds(bo, BB)]], gbuf, sem)

        def wait_gather(gbuf, sem):
            pltpu.make_async_copy(
                table_hbm.at[pl.ds(0, BB)], gbuf, sem).wait()

        def transpose(gbuf, tbuf):
            @pl.loop(0, BB, step=16, unroll=2)
            def _(r0):
                rows = lanes + r0
                for f0 in range(0, D, 16):
                    for d in range(16):
                        cols = diag[d] + f0
                        vals = plsc.load_gather(gbuf, [rows, cols])
                        plsc.store_scatter(tbuf, [cols, rows], vals)

        def fire_write(j, tbuf, sem):
            sq, bo = split(j)
            pltpu.async_copy(
                tbuf, out_hbm.at[sq, :, pl.ds(col0 + bo, BB)], sem)

        def wait_write(tbuf, sem):
            pltpu.make_async_copy(
                tbuf, out_hbm.at[0, pl.ds(0, D), pl.ds(0, BB)], sem).wait()

        fire_gather(0, gbuf_a, gsem_a)

        @pl.loop(0, n_blocks, step=2)
        def body(ja):
            jb = ja + 1
            wait_gather(gbuf_a, gsem_a)
            fire_gather(jb, gbuf_b, gsem_b)

            @pl.when(ja > 0)
            def _():
                wait_write(tbuf_a, wsem_a)

            transpose(gbuf_a, tbuf_a)
            fire_write(ja, tbuf_a, wsem_a)
            wait_gather(gbuf_b, gsem_b)

            @pl.when(jb + 1 < n_blocks)
            def _():
                fire_gather(ja + 2, gbuf_a, gsem_a)

            @pl.when(ja > 0)
            def _():
                wait_write(tbuf_b, wsem_b)

            transpose(gbuf_b, tbuf_b)
            fire_write(jb, tbuf_b, wsem_b)

        wait_write(tbuf_a, wsem_a)
        wait_write(tbuf_b, wsem_b)

    return emb


def kernel(input_ids, table):
    b, s = input_ids.shape
    v = table.shape[0]
    table2 = _build_prep(v)(table.T)
    out = _build_lookup(b, s, v)(input_ids.T, table2)
    return out.transpose(2, 0, 1)
